# baseline (speedup 1.0000x reference)
import jax
import jax.numpy as jnp
import numpy as np
from jax.experimental import pallas as pl
from jax.experimental.pallas import tpu as pltpu

_B, _C, _H, _W, _D = 16, 512, 14, 14, 8192
_NPIX = _B * _H * _W
_D1, _D2 = 64, 128
_PA = 784
_PB = 32

_CompilerParams = getattr(pltpu, "CompilerParams", None) or pltpu.TPUCompilerParams


def _trig():
    i64 = np.arange(_D1, dtype=np.float64)
    i128 = np.arange(_D2, dtype=np.float64)
    tp = 2.0 * np.pi
    f32 = np.float32
    c128 = np.cos(tp * np.outer(i128, i128) / _D2).astype(f32)
    s128 = np.sin(tp * np.outer(i128, i128) / _D2).astype(f32)
    twi = tp * np.outer(i64, i128) / _D
    twir = np.ascontiguousarray(np.broadcast_to(
        np.cos(twi).astype(f32)[:, None, :], (_D1, _PB, _D2)))
    twii = np.ascontiguousarray(np.broadcast_to(
        np.sin(twi).astype(f32)[:, None, :], (_D1, _PB, _D2)))
    c64i = np.cos(tp * np.outer(i64, i64) / _D1).astype(f32)
    s64i = np.sin(tp * np.outer(i64, i64) / _D1).astype(f32)
    return c128, s128, twir, twii, c64i, s64i


_TRIG = _trig()


def _proj_kernel(x_ref, w_ref, b_ref):
    xb = x_ref[0]
    for a in range(_D // 256):
        v = jnp.dot(xb, w_ref[0, :, a * 256:(a + 1) * 256],
                    preferred_element_type=jnp.float32)
        v = v.astype(jnp.bfloat16)
        b_ref[0, 2 * a] = v[:, :128]
        b_ref[0, 2 * a + 1] = v[:, 128:]


def _fft_kernel(b_ref, c128, s128, twir, twii, c64i, s64i, out_ref):
    f32 = jnp.float32
    cm = c128[...]
    sm = s128[...]

    def dot3(t, m):
        return jnp.einsum('kpm,mn->kpn', t, m, preferred_element_type=f32)

    b1re = b_ref[0].astype(f32)
    b1im = b_ref[1].astype(f32)
    b2re = b_ref[2].astype(f32)
    b2im = b_ref[3].astype(f32)
    y1re = dot3(b1re, cm) + dot3(b1im, sm)
    y1im = dot3(b1im, cm) - dot3(b1re, sm)
    y2re = dot3(b2re, cm) + dot3(b2im, sm)
    y2im = dot3(b2im, cm) - dot3(b2re, sm)
    fre = y1re * y2re - y1im * y2im
    fim = y1re * y2im + y1im * y2re
    p1re = dot3(fre, cm) - dot3(fim, sm)
    p1im = dot3(fre, sm) + dot3(fim, cm)
    p2re = p1re * twir[...] - p1im * twii[...]
    p2im = p1re * twii[...] + p1im * twir[...]
    outv = (jnp.einsum('na,apm->npm', c64i[...], p2re, preferred_element_type=f32)
            - jnp.einsum('na,apm->npm', s64i[...], p2im, preferred_element_type=f32))
    out_ref[...] = outv


def kernel(x1, x2, S1, S2):
    bf16 = jnp.bfloat16
    f32 = jnp.float32
    xs = (jnp.stack([x1.reshape(_B, _C, _H * _W), x2.reshape(_B, _C, _H * _W)])
          .transpose(0, 1, 3, 2).reshape(2, _NPIX, _C).astype(bf16))

    k64 = jnp.arange(_D1, dtype=f32)
    lane = jnp.arange(_D2, dtype=jnp.int32)

    def fold(S):
        s = jnp.sum(S, axis=1)
        h = jnp.argmax(jnp.abs(S), axis=1).astype(jnp.int32)
        ph = (2.0 * np.pi / _D) * h.astype(f32)[:, None] * k64[None, :]
        onehot = (lane[None, :] == (h % _D2)[:, None]).astype(f32)
        wre = (jnp.cos(ph) * s[:, None])[:, :, None] * onehot[:, None, :]
        wim = (-jnp.sin(ph) * s[:, None])[:, :, None] * onehot[:, None, :]
        return wre, wim

    w1re, w1im = fold(S1)
    w2re, w2im = fold(S2)
    wq = (jnp.stack([w1re, w1im, w2re, w2im])
          .reshape(4, _C, _D).astype(bf16))

    bq = pl.pallas_call(
        _proj_kernel,
        grid=(4, _NPIX // _PA),
        in_specs=[
            pl.BlockSpec((1, _PA, _C), lambda w, j: (w // 2, j, 0)),
            pl.BlockSpec((1, _C, _D), lambda w, j: (w, 0, 0)),
        ],
        out_specs=pl.BlockSpec((1, _D1, _PA, _D2), lambda w, j: (w, 0, j, 0)),
        out_shape=jax.ShapeDtypeStruct((4, _D1, _NPIX, _D2), bf16),
        compiler_params=_CompilerParams(
            dimension_semantics=("parallel", "parallel"),
            vmem_limit_bytes=100 * 1024 * 1024,
        ),
    )(xs, wq)

    trig = [jnp.asarray(t) for t in _TRIG]
    const_specs = [pl.BlockSpec(t.shape, lambda j, n=t.ndim: (0,) * n)
                   for t in trig]

    outv = pl.pallas_call(
        _fft_kernel,
        grid=(_NPIX // _PB,),
        in_specs=[pl.BlockSpec((4, _D1, _PB, _D2), lambda j: (0, 0, j, 0))]
        + const_specs,
        out_specs=pl.BlockSpec((_D1, _PB, _D2), lambda j: (0, j, 0)),
        out_shape=jax.ShapeDtypeStruct((_D1, _NPIX, _D2), jnp.float32),
        compiler_params=_CompilerParams(
            dimension_semantics=("parallel",),
            vmem_limit_bytes=100 * 1024 * 1024,
        ),
    )(bq, *trig)

    return outv.transpose(1, 0, 2).reshape(_B, _H, _W, _D)
